# baseline (speedup 1.0000x reference)
import functools

import jax
import jax.numpy as jnp
from jax import lax
from jax.experimental import pallas as pl
from jax.experimental.pallas import tpu as pltpu
from jax.experimental.pallas import tpu_sc as plsc

B = 16384
D = 128
H = 256

_NSPLIT = 4
_BS = B // _NSPLIT

_NC = 2
_NS = 16
_NW = _NC * _NS
_CHUNK = 128


def _gather_body(bs, off0, uidx, iidx, utab, itab, uout, iout,
                 idx0, idx1, rows0, rows1, sem0, sem1):
    bpw = bs // _NW
    nch = bpw // _CHUNK
    wid = lax.axis_index("s") * _NC + lax.axis_index("c")
    base_w = wid * bpw
    idx_v = (idx0, idx1)
    rows_v = (rows0, rows1)
    sems = (sem0, sem1)
    jobs = []
    for j in range(nch):
        jobs.append((uidx, utab, uout, j * _CHUNK))
    for j in range(nch):
        jobs.append((iidx, itab, iout, j * _CHUNK))

    copies = [None, None]
    for k, (src_idx, tab, out, off) in enumerate(jobs):
        s = k % 2
        if copies[s] is not None:
            copies[s][0].wait()
            pltpu.sync_copy(rows_v[s], copies[s][1])
            copies[s] = None
        b = base_w + off
        pltpu.sync_copy(src_idx.at[pl.ds(off0 + b, _CHUNK)], idx_v[s])
        cp = pltpu.async_copy(tab.at[idx_v[s]], rows_v[s], sems[s])
        copies[s] = (cp, out.at[pl.ds(b, _CHUNK)])
    for s in range(2):
        if copies[s] is not None:
            copies[s][0].wait()
            pltpu.sync_copy(rows_v[s], copies[s][1])


@functools.lru_cache(maxsize=None)
def _make_gather(bs, off0):
    return functools.partial(
        pl.kernel,
        mesh=plsc.VectorSubcoreMesh(core_axis_name="c", subcore_axis_name="s"),
        out_type=[
            jax.ShapeDtypeStruct((bs, D), jnp.float32),
            jax.ShapeDtypeStruct((bs, D), jnp.float32),
        ],
        scratch_types=[
            pltpu.VMEM((_CHUNK,), jnp.int32),
            pltpu.VMEM((_CHUNK,), jnp.int32),
            pltpu.VMEM((_CHUNK, D), jnp.float32),
            pltpu.VMEM((_CHUNK, D), jnp.float32),
            pltpu.SemaphoreType.DMA,
            pltpu.SemaphoreType.DMA,
        ],
    )(functools.partial(_gather_body, bs, off0))


_BM = 2048


def _ln_relu(x, g, b):
    m = jnp.mean(x, axis=-1, keepdims=True)
    xc = x - m
    v = jnp.mean(xc * xc, axis=-1, keepdims=True)
    return jnp.maximum(xc * lax.rsqrt(v + 1e-5) * g + b, 0.0)


def _mlp_body(ue_ref, ie_ref,
              uW1r, ub1r, ug1r, ube1r, uW2r, ub2r, ug2r, ube2r,
              iW1r, ib1r, ig1r, ibe1r, iW2r, ib2r, ig2r, ibe2r,
              fW1r, fb1r, fg1r, fbe1r, fW2r, fb2r,
              out_ref):
    dot = functools.partial(jnp.dot, preferred_element_type=jnp.float32)
    ux = _ln_relu(dot(ue_ref[...], uW1r[...]) + ub1r[...], ug1r[...], ube1r[...])
    ux = _ln_relu(dot(ux, uW2r[...]) + ub2r[...], ug2r[...], ube2r[...])
    ix = _ln_relu(dot(ie_ref[...], iW1r[...]) + ib1r[...], ig1r[...], ibe1r[...])
    ix = _ln_relu(dot(ix, iW2r[...]) + ib2r[...], ig2r[...], ibe2r[...])
    h = dot(ux, fW1r[:H, :]) + dot(ix, fW1r[H:, :]) + fb1r[...]
    h = _ln_relu(h, fg1r[...], fbe1r[...])
    z = dot(h, fW2r[...]) + fb2r[...]
    out_ref[...] = jax.nn.sigmoid(z[:, 0])


def _full(shape):
    return pl.BlockSpec(shape, lambda i: (0,) * len(shape))


def _mlp(ue, ie, *ws):
    bs = ue.shape[0]
    bm = min(_BM, bs)
    in_specs = [
        pl.BlockSpec((bm, D), lambda i: (i, 0)),
        pl.BlockSpec((bm, D), lambda i: (i, 0)),
    ] + [_full(w.shape) for w in ws]
    return pl.pallas_call(
        _mlp_body,
        grid=(bs // bm,),
        in_specs=in_specs,
        out_specs=pl.BlockSpec((bm,), lambda i: (i,)),
        out_shape=jax.ShapeDtypeStruct((bs,), jnp.float32),
        compiler_params=pltpu.CompilerParams(
            dimension_semantics=("parallel",)),
    )(ue, ie, *ws)


def kernel(user_indices, item_indices, user_table, item_table,
           uW1, ub1, ug1, ube1, uW2, ub2, ug2, ube2,
           iW1, ib1, ig1, ibe1, iW2, ib2, ig2, ibe2,
           fW1, fb1, fg1, fbe1, fW2, fb2):
    r = lambda v: v.reshape(1, -1)
    ws = (uW1, r(ub1), r(ug1), r(ube1), uW2, r(ub2), r(ug2), r(ube2),
          iW1, r(ib1), r(ig1), r(ibe1), iW2, r(ib2), r(ig2), r(ibe2),
          fW1, r(fb1), r(fg1), r(fbe1), fW2, r(fb2))
    gathered = [
        _make_gather(_BS, s * _BS)(user_indices, item_indices,
                                   user_table, item_table)
        for s in range(_NSPLIT)
    ]
    outs = [_mlp(ue, ie, *ws) for ue, ie in gathered]
    return jnp.concatenate(outs) if len(outs) > 1 else outs[0]


# speedup vs baseline: 1.0278x; 1.0278x over previous
import functools

import jax
import jax.numpy as jnp
from jax import lax
from jax.experimental import pallas as pl
from jax.experimental.pallas import tpu as pltpu
from jax.experimental.pallas import tpu_sc as plsc

B = 16384
D = 128
H = 256

_NSPLIT = 2
_BS = B // _NSPLIT

_NC = 2
_NS = 16
_NW = _NC * _NS
_CHUNK = 128


def _gather_body(bs, off0, uidx, iidx, utab, itab, uout, iout,
                 idx0, idx1, rows0, rows1, sem0, sem1):
    bpw = bs // _NW
    nch = bpw // _CHUNK
    wid = lax.axis_index("s") * _NC + lax.axis_index("c")
    base_w = wid * bpw
    idx_v = (idx0, idx1)
    rows_v = (rows0, rows1)
    sems = (sem0, sem1)
    jobs = []
    for j in range(nch):
        jobs.append((uidx, utab, uout, j * _CHUNK))
    for j in range(nch):
        jobs.append((iidx, itab, iout, j * _CHUNK))

    copies = [None, None]
    for k, (src_idx, tab, out, off) in enumerate(jobs):
        s = k % 2
        if copies[s] is not None:
            copies[s][0].wait()
            pltpu.sync_copy(rows_v[s], copies[s][1])
            copies[s] = None
        b = base_w + off
        pltpu.sync_copy(src_idx.at[pl.ds(off0 + b, _CHUNK)], idx_v[s])
        cp = pltpu.async_copy(tab.at[idx_v[s]], rows_v[s], sems[s])
        copies[s] = (cp, out.at[pl.ds(b, _CHUNK)])
    for s in range(2):
        if copies[s] is not None:
            copies[s][0].wait()
            pltpu.sync_copy(rows_v[s], copies[s][1])


@functools.lru_cache(maxsize=None)
def _make_gather(bs, off0):
    return functools.partial(
        pl.kernel,
        mesh=plsc.VectorSubcoreMesh(core_axis_name="c", subcore_axis_name="s"),
        out_type=[
            jax.ShapeDtypeStruct((bs, D), jnp.float32),
            jax.ShapeDtypeStruct((bs, D), jnp.float32),
        ],
        scratch_types=[
            pltpu.VMEM((_CHUNK,), jnp.int32),
            pltpu.VMEM((_CHUNK,), jnp.int32),
            pltpu.VMEM((_CHUNK, D), jnp.float32),
            pltpu.VMEM((_CHUNK, D), jnp.float32),
            pltpu.SemaphoreType.DMA,
            pltpu.SemaphoreType.DMA,
        ],
    )(functools.partial(_gather_body, bs, off0))


_BM = 2048


def _ln_relu(x, g, b):
    m = jnp.mean(x, axis=-1, keepdims=True)
    xc = x - m
    v = jnp.mean(xc * xc, axis=-1, keepdims=True)
    return jnp.maximum(xc * lax.rsqrt(v + 1e-5) * g + b, 0.0)


def _mlp_body(ue_ref, ie_ref,
              uW1r, ub1r, ug1r, ube1r, uW2r, ub2r, ug2r, ube2r,
              iW1r, ib1r, ig1r, ibe1r, iW2r, ib2r, ig2r, ibe2r,
              fW1r, fb1r, fg1r, fbe1r, fW2r, fb2r,
              out_ref):
    dot = functools.partial(jnp.dot, preferred_element_type=jnp.float32)
    ux = _ln_relu(dot(ue_ref[...], uW1r[...]) + ub1r[...], ug1r[...], ube1r[...])
    ux = _ln_relu(dot(ux, uW2r[...]) + ub2r[...], ug2r[...], ube2r[...])
    ix = _ln_relu(dot(ie_ref[...], iW1r[...]) + ib1r[...], ig1r[...], ibe1r[...])
    ix = _ln_relu(dot(ix, iW2r[...]) + ib2r[...], ig2r[...], ibe2r[...])
    h = dot(ux, fW1r[:H, :]) + dot(ix, fW1r[H:, :]) + fb1r[...]
    h = _ln_relu(h, fg1r[...], fbe1r[...])
    z = dot(h, fW2r[...]) + fb2r[...]
    out_ref[...] = jax.nn.sigmoid(z[:, 0])


def _full(shape):
    return pl.BlockSpec(shape, lambda i: (0,) * len(shape))


def _mlp(ue, ie, *ws):
    bs = ue.shape[0]
    bm = min(_BM, bs)
    in_specs = [
        pl.BlockSpec((bm, D), lambda i: (i, 0)),
        pl.BlockSpec((bm, D), lambda i: (i, 0)),
    ] + [_full(w.shape) for w in ws]
    return pl.pallas_call(
        _mlp_body,
        grid=(bs // bm,),
        in_specs=in_specs,
        out_specs=pl.BlockSpec((bm,), lambda i: (i,)),
        out_shape=jax.ShapeDtypeStruct((bs,), jnp.float32),
        compiler_params=pltpu.CompilerParams(
            dimension_semantics=("parallel",)),
    )(ue, ie, *ws)


def kernel(user_indices, item_indices, user_table, item_table,
           uW1, ub1, ug1, ube1, uW2, ub2, ug2, ube2,
           iW1, ib1, ig1, ibe1, iW2, ib2, ig2, ibe2,
           fW1, fb1, fg1, fbe1, fW2, fb2):
    r = lambda v: v.reshape(1, -1)
    ws = (uW1, r(ub1), r(ug1), r(ube1), uW2, r(ub2), r(ug2), r(ube2),
          iW1, r(ib1), r(ig1), r(ibe1), iW2, r(ib2), r(ig2), r(ibe2),
          fW1, r(fb1), r(fg1), r(fbe1), fW2, r(fb2))
    gathered = [
        _make_gather(_BS, s * _BS)(user_indices, item_indices,
                                   user_table, item_table)
        for s in range(_NSPLIT)
    ]
    outs = [_mlp(ue, ie, *ws) for ue, ie in gathered]
    return jnp.concatenate(outs) if len(outs) > 1 else outs[0]
